# baseline (speedup 1.0000x reference)
import jax
import jax.numpy as jnp
from jax.experimental import pallas as pl
from jax.experimental.pallas import tpu as pltpu

D_MODEL = 128
SEQ = 64
BATCH_BLOCK = 128


def _body(inds_ref, x_ref, pe_ref, o_ref, fp_ref):
    @pl.when(pl.program_id(0) == 0)
    def _():
        def gather_row(j, _):
            idx = inds_ref[j]
            fp_ref[pl.ds(j, 1), :] = pe_ref[pl.ds(idx, 1), :]
            return 0

        jax.lax.fori_loop(0, SEQ, gather_row, 0)

    o_ref[...] = x_ref[...] + fp_ref[...][None, :, :]


def kernel(x, x_node_inds, pe):
    batch = x.shape[0]
    nb = batch // BATCH_BLOCK
    inds = x_node_inds.astype(jnp.int32)
    pe64 = pe[:SEQ]

    grid_spec = pltpu.PrefetchScalarGridSpec(
        num_scalar_prefetch=1,
        grid=(nb,),
        in_specs=[
            pl.BlockSpec((BATCH_BLOCK, SEQ, D_MODEL), lambda i, inds_ref: (i, 0, 0)),
            pl.BlockSpec((SEQ, D_MODEL), lambda i, inds_ref: (0, 0)),
        ],
        out_specs=pl.BlockSpec((BATCH_BLOCK, SEQ, D_MODEL), lambda i, inds_ref: (i, 0, 0)),
        scratch_shapes=[pltpu.VMEM((SEQ, D_MODEL), jnp.float32)],
    )

    return pl.pallas_call(
        _body,
        grid_spec=grid_spec,
        out_shape=jax.ShapeDtypeStruct(x.shape, x.dtype),
        compiler_params=pltpu.CompilerParams(
            dimension_semantics=("arbitrary",),
        ),
    )(inds, x, pe64)


# speedup vs baseline: 1.0225x; 1.0225x over previous
import jax
import jax.numpy as jnp
from jax.experimental import pallas as pl
from jax.experimental.pallas import tpu as pltpu

D_MODEL = 128
SEQ = 64
BATCH_BLOCK = 256


def _body(inds_ref, x_ref, pe_ref, o_ref, fp_ref):
    @pl.when(pl.program_id(0) == 0)
    def _():
        def gather_row(j, _):
            idx = inds_ref[j]
            fp_ref[pl.ds(j, 1), :] = pe_ref[pl.ds(idx, 1), :]
            return 0

        jax.lax.fori_loop(0, SEQ, gather_row, 0)

    o_ref[...] = x_ref[...] + fp_ref[...][None, :, :]


def kernel(x, x_node_inds, pe):
    batch = x.shape[0]
    nb = batch // BATCH_BLOCK
    inds = x_node_inds.astype(jnp.int32)
    pe64 = pe[:SEQ]

    grid_spec = pltpu.PrefetchScalarGridSpec(
        num_scalar_prefetch=1,
        grid=(nb,),
        in_specs=[
            pl.BlockSpec((BATCH_BLOCK, SEQ, D_MODEL), lambda i, inds_ref: (i, 0, 0)),
            pl.BlockSpec((SEQ, D_MODEL), lambda i, inds_ref: (0, 0)),
        ],
        out_specs=pl.BlockSpec((BATCH_BLOCK, SEQ, D_MODEL), lambda i, inds_ref: (i, 0, 0)),
        scratch_shapes=[pltpu.VMEM((SEQ, D_MODEL), jnp.float32)],
    )

    return pl.pallas_call(
        _body,
        grid_spec=grid_spec,
        out_shape=jax.ShapeDtypeStruct(x.shape, x.dtype),
        compiler_params=pltpu.CompilerParams(
            dimension_semantics=("arbitrary",),
        ),
    )(inds, x, pe64)
